# baseline (speedup 1.0000x reference)
import functools

import jax
import jax.numpy as jnp
from jax import lax
from jax.experimental import pallas as pl
from jax.experimental.pallas import tpu as pltpu
from jax.experimental.pallas import tpu_sc as plsc

D = 768
L = 16
J = D // L
NC, NS = 2, 16
NW = NC * NS
C = 32
EPS = 1e-12


def _rsqrt16(x):
    i = lax.bitcast_convert_type(x, jnp.int32)
    y = lax.bitcast_convert_type(jnp.int32(0x5F3759DF) - (i >> 1),
                                 jnp.float32)
    for _ in range(3):
        y = y * (1.5 - 0.5 * x * y * y)
    return y


_GDN = lax.GatherDimensionNumbers(
    offset_dims=(), collapsed_slice_dims=(0,), start_index_map=(0,))


def _gather16(vec, idx):
    return lax.gather(vec, idx.reshape(L, 1), _GDN, slice_sizes=(1,),
                      mode=lax.GatherScatterMode.PROMISE_IN_BOUNDS)


def _splat_lane(vec, lane):
    return _gather16(vec, jnp.full((L,), lane, jnp.int32))


def _lane_sum(v):
    iota = lax.iota(jnp.int32, L)
    for sh in (8, 4, 2, 1):
        v = v + _gather16(v, (iota + sh) & (L - 1))
    return v


def _make_sc_kernel(batch, seq):
    n_tokens = batch * seq
    s_per_w = seq // NW
    n_chunks = batch * s_per_w // C
    hpb = s_per_w // C
    mesh = plsc.VectorSubcoreMesh(core_axis_name="c", subcore_axis_name="s")

    @functools.partial(
        pl.kernel,
        out_type=jax.ShapeDtypeStruct((n_tokens, D), jnp.float32),
        mesh=mesh,
        scratch_types=[
            pltpu.VMEM((n_chunks, C), jnp.int32),
            pltpu.VMEM((n_chunks, C), jnp.int32),
            pltpu.VMEM((C, D), jnp.float32),
            pltpu.VMEM((C, D), jnp.float32),
            pltpu.VMEM((s_per_w, D), jnp.float32),
            pltpu.VMEM((D,), jnp.float32),
            pltpu.VMEM((D,), jnp.float32),
            pltpu.VMEM((D,), jnp.float32),
            pltpu.VMEM((D,), jnp.float32),
            pltpu.VMEM((2, D), jnp.float32),
            pltpu.SemaphoreType.DMA,
            pltpu.SemaphoreType.DMA,
            pltpu.SemaphoreType.DMA,
            pltpu.SemaphoreType.DMA,
            pltpu.SemaphoreType.DMA,
        ],
    )
    def sc_kernel(ids_hbm, tid_hbm, word_hbm, type_hbm, pos_hbm, lnw_hbm,
                  lnb_hbm, out_hbm, idsbuf, tidsbuf, wbuf0, wbuf1, pbuf,
                  t0buf, dbuf, wlbuf, blbuf, typebuf, gsem0, gsem1, osem0,
                  osem1, isem):
        cid = lax.axis_index("c")
        sid = lax.axis_index("s")
        wid = sid * NC + cid
        s0 = wid * s_per_w

        wbufs = [wbuf0, wbuf1]
        gsems = [gsem0, gsem1]
        osems = [osem0, osem1]

        pro = []
        for b in range(batch):
            src = pl.ds(b * (seq // C) + wid * hpb, hpb)
            dst = pl.ds(b * hpb, hpb)
            pro.append(pltpu.async_copy(ids_hbm.at[src], idsbuf.at[dst],
                                        isem))
            pro.append(pltpu.async_copy(tid_hbm.at[src], tidsbuf.at[dst],
                                        isem))
        pro.append(pltpu.async_copy(pos_hbm.at[pl.ds(s0, s_per_w)], pbuf,
                                    isem))
        pro.append(pltpu.async_copy(type_hbm, typebuf, isem))
        pro.append(pltpu.async_copy(lnw_hbm, wlbuf, isem))
        pro.append(pltpu.async_copy(lnb_hbm, blbuf, isem))
        for d in pro:
            d.wait()

        def prep(j, _):
            sl = pl.ds(j * L, L)
            t0 = typebuf[0, sl]
            t1 = typebuf[1, sl]
            t0buf[sl] = t0
            dbuf[sl] = t1 - t0
            return 0

        lax.fori_loop(0, J, prep, 0)

        def fold_t0(r, _):
            @plsc.parallel_loop(0, J, unroll=4)
            def _(j):
                sl = pl.ds(j * L, L)
                pbuf[r, sl] = pbuf[r, sl] + t0buf[sl]
            return 0

        lax.fori_loop(0, s_per_w, fold_t0, 0)

        def out_row0(k):
            b, h = divmod(k, hpb)
            return b * seq + s0 + h * C

        def compute_chunk(k, wbuf):
            h = k % hpb

            def pair_body(tvecf0, tvecf1, rr, _):
                r0 = rr
                r1 = L + rr
                tidf0 = _splat_lane(tvecf0, rr)
                tidf1 = _splat_lane(tvecf1, rr)
                pr0 = h * C + r0
                pr1 = h * C + r1
                zero = jnp.zeros((L,), jnp.float32)

                @plsc.parallel_loop(0, J // 2, carry=(zero,) * 8, unroll=2)
                def acc(ji, carry):
                    s0a, q0a, s0b, q0b, s1a, q1a, s1b, q1b = carry
                    ja = pl.ds((2 * ji) * L, L)
                    jb = pl.ds((2 * ji + 1) * L, L)
                    da = dbuf[ja]
                    db = dbuf[jb]
                    va0 = wbuf[r0, ja] + pbuf[pr0, ja] + tidf0 * da
                    vb0 = wbuf[r0, jb] + pbuf[pr0, jb] + tidf0 * db
                    va1 = wbuf[r1, ja] + pbuf[pr1, ja] + tidf1 * da
                    vb1 = wbuf[r1, jb] + pbuf[pr1, jb] + tidf1 * db
                    wbuf[r0, ja] = va0
                    wbuf[r0, jb] = vb0
                    wbuf[r1, ja] = va1
                    wbuf[r1, jb] = vb1
                    return (s0a + va0, q0a + va0 * va0,
                            s0b + vb0, q0b + vb0 * vb0,
                            s1a + va1, q1a + va1 * va1,
                            s1b + vb1, q1b + vb1 * vb1)

                s0a, q0a, s0b, q0b, s1a, q1a, s1b, q1b = acc
                mean0 = _lane_sum(s0a + s0b) * (1.0 / D)
                var0 = _lane_sum(q0a + q0b) * (1.0 / D) - mean0 * mean0
                mean1 = _lane_sum(s1a + s1b) * (1.0 / D)
                var1 = _lane_sum(q1a + q1b) * (1.0 / D) - mean1 * mean1
                rstd0 = _rsqrt16(var0 + EPS)
                rstd1 = _rsqrt16(var1 + EPS)

                @plsc.parallel_loop(0, J, unroll=2)
                def _(j):
                    sl = pl.ds(j * L, L)
                    wl = wlbuf[sl]
                    bl = blbuf[sl]
                    v0 = (wbuf[r0, sl] - mean0) * rstd0
                    wbuf[r0, sl] = v0 * wl + bl
                    v1 = (wbuf[r1, sl] - mean1) * rstd1
                    wbuf[r1, sl] = v1 * wl + bl

                return 0

            tvecf0 = tidsbuf[k, pl.ds(0, L)].astype(jnp.float32)
            tvecf1 = tidsbuf[k, pl.ds(L, L)].astype(jnp.float32)
            lax.fori_loop(0, L, functools.partial(pair_body, tvecf0, tvecf1),
                          0)

        gdesc = [None, None]
        odesc = [None, None]
        gdesc[0] = pltpu.async_copy(word_hbm.at[idsbuf.at[0]], wbufs[0],
                                    gsems[0])
        for k in range(n_chunks):
            buf = k % 2
            nb = buf ^ 1
            if k + 1 < n_chunks:
                if odesc[nb] is not None:
                    odesc[nb].wait()
                    odesc[nb] = None
                gdesc[nb] = pltpu.async_copy(
                    word_hbm.at[idsbuf.at[k + 1]], wbufs[nb], gsems[nb])
            gdesc[buf].wait()
            compute_chunk(k, wbufs[buf])
            odesc[buf] = pltpu.async_copy(
                wbufs[buf], out_hbm.at[pl.ds(out_row0(k), C)], osems[buf])
        odesc[0].wait()
        odesc[1].wait()

    return sc_kernel


def kernel(input_ids, token_ids, word_emb, type_emb, pos_emb, ln_weight,
           ln_bias):
    batch, seq = input_ids.shape
    n = batch * seq
    ids = input_ids.reshape(n // C, C).astype(jnp.int32)
    tids = token_ids.reshape(n // C, C).astype(jnp.int32)
    sc = _make_sc_kernel(batch, seq)
    out = sc(ids, tids, word_emb, type_emb, pos_emb, ln_weight, ln_bias)
    return out.reshape(batch, seq, D)
